# Initial kernel scaffold; baseline (speedup 1.0000x reference)
import jax
import jax.numpy as jnp
from jax.experimental import pallas as pl


def kernel(x):
    raise NotImplementedError("write your pallas kernel here")


# speedup vs baseline: 6.5068x; 6.5068x over previous
import functools

import jax
from jax import lax
import jax.numpy as jnp
from jax.experimental import pallas as pl
from jax.experimental.pallas import tpu as pltpu
from jax.experimental.pallas import tpu_sc as plsc

_C = 64
_T = 16384
_B = 8
_OUT_T = 32768
_L = 16

_OFFSETS = (1220, 18, 1207, 3265, 653, 3435, 2433, 2343)

_MESH = plsc.VectorSubcoreMesh(core_axis_name="c", subcore_axis_name="s")


@functools.partial(
    pl.kernel,
    mesh=_MESH,
    out_type=jax.ShapeDtypeStruct((_B * _C * _OUT_T,), jnp.float32),
    scratch_types=[
        pltpu.VMEM((2 * _T,), jnp.float32),
        pltpu.VMEM((_T,), jnp.float32),
    ],
    compiler_params=pltpu.CompilerParams(needs_layout_passes=False),
)
def _jitter_sc(x_hbm, out_hbm, xx, rot):
    wid = lax.axis_index("s") * 2 + lax.axis_index("c")
    lanes = lax.iota(jnp.int32, _L)
    for j in range(2):
        ch = wid * 2 + j
        src_base = pl.multiple_of(ch * _T, _T)
        pltpu.sync_copy(x_hbm.at[pl.ds(src_base, _T)], xx.at[pl.ds(0, _T)])
        pltpu.sync_copy(x_hbm.at[pl.ds(src_base, _T)], xx.at[pl.ds(_T, _T)])
        for b in range(_B):
            off = _OFFSETS[b]

            def body(i, _, off=off):
                base = pl.multiple_of(i * _L, _L)
                vals = plsc.load_gather(xx, [base + (off + lanes)])
                rot[pl.ds(base, _L)] = vals
                return _

            lax.fori_loop(0, _T // _L, body, None)
            dst = pl.multiple_of((b * _C + ch) * _OUT_T, _OUT_T)
            pltpu.sync_copy(rot, out_hbm.at[pl.ds(dst, _T)])
            pltpu.sync_copy(rot, out_hbm.at[pl.ds(dst + _T, _T)])


def kernel(x):
    out = _jitter_sc(x.reshape(_C * _T))
    return out.reshape(_B, _C, _OUT_T)


# speedup vs baseline: 10.0617x; 1.5463x over previous
import functools

import jax
from jax import lax
import jax.numpy as jnp
from jax.experimental import pallas as pl
from jax.experimental.pallas import tpu as pltpu
from jax.experimental.pallas import tpu_sc as plsc

_C = 64
_T = 16384
_B = 8
_OUT_T = 32768
_L = 16

_OFFSETS = (1220, 18, 1207, 3265, 653, 3435, 2433, 2343)

_MESH = plsc.VectorSubcoreMesh(core_axis_name="c", subcore_axis_name="s")


@functools.partial(
    pl.kernel,
    mesh=_MESH,
    out_type=jax.ShapeDtypeStruct((_B * _C * _OUT_T,), jnp.float32),
    scratch_types=[
        pltpu.VMEM((2 * _T,), jnp.float32),
        pltpu.VMEM((_T,), jnp.float32),
        pltpu.VMEM((_T,), jnp.float32),
        pltpu.SemaphoreType.DMA,
        pltpu.SemaphoreType.DMA,
        pltpu.SemaphoreType.DMA,
    ],
    compiler_params=pltpu.CompilerParams(needs_layout_passes=False),
)
def _jitter_sc(x_hbm, out_hbm, xx, rot0, rot1, sem0, sem1, sem_in):
    wid = lax.axis_index("s") * 2 + lax.axis_index("c")
    lanes = lax.iota(jnp.int32, _L)
    rots = (rot0, rot1)
    sems = (sem0, sem1)
    inflight = [[], []]
    for j in range(2):
        ch = wid * 2 + j
        src_base = pl.multiple_of(ch * _T, _T)
        in0 = pltpu.async_copy(
            x_hbm.at[pl.ds(src_base, _T)], xx.at[pl.ds(0, _T)], sem_in)
        in1 = pltpu.async_copy(
            x_hbm.at[pl.ds(src_base, _T)], xx.at[pl.ds(_T, _T)], sem_in)
        in0.wait()
        in1.wait()
        for b in range(_B):
            off = _OFFSETS[b]
            slot = b % 2
            rot = rots[slot]
            for d in inflight[slot]:
                d.wait()
            inflight[slot] = []

            def body(i, rot=rot, off=off):
                for u in range(8):
                    base = pl.multiple_of(i + u * _L, _L)
                    vals = plsc.load_gather(xx, [base + (off + lanes)])
                    rot[pl.ds(base, _L)] = vals

            plsc.parallel_loop(0, _T, _L * 8, unroll=8)(body)

            dst = pl.multiple_of((b * _C + ch) * _OUT_T, _OUT_T)
            d0 = pltpu.async_copy(rot, out_hbm.at[pl.ds(dst, _T)], sems[slot])
            d1 = pltpu.async_copy(
                rot, out_hbm.at[pl.ds(dst + _T, _T)], sems[slot])
            inflight[slot] = [d0, d1]
    for descs in inflight:
        for d in descs:
            d.wait()


def kernel(x):
    out = _jitter_sc(x.reshape(_C * _T))
    return out.reshape(_B, _C, _OUT_T)


# speedup vs baseline: 19.4235x; 1.9304x over previous
import jax
from jax import lax
import jax.numpy as jnp
from jax.experimental import pallas as pl
from jax.experimental.pallas import tpu as pltpu
from jax.experimental.pallas import tpu_sc as plsc

_C = 64
_T = 16384
_B = 8
_OUT_T = 32768
_L = 16
_W = 2048
_NSUB = 17

_OFFSETS = (1220, 18, 1207, 3265, 653, 3435, 2433, 2343)

_MESH = plsc.VectorSubcoreMesh(core_axis_name="c", subcore_axis_name="s")


def _sel_offset(b):
    off = jnp.int32(_OFFSETS[0])
    for i in range(1, _B):
        off = jnp.where(b == i, jnp.int32(_OFFSETS[i]), off)
    return off


def _jitter_sc(x_hbm, out_hbm, xxc0, xxc1, rot0, rot1,
               semi0, semi1, semo0, semo1):
    wid = lax.axis_index("s") * 2 + lax.axis_index("c")
    o8 = pl.multiple_of((wid >> 2) * 8, 8)
    q = wid & 3
    lanes = lax.iota(jnp.int32, _L)
    xxcs, semis = (xxc0, xxc1), (semi0, semi1)
    rots, semos = (rot0, rot1), (semo0, semo1)

    def issue_in(batch, j2, slot):
        off = _sel_offset(batch)
        base = ((off >> 7) << 7) + (4 * q + 2 * j2) * 1024
        xxc, sem = xxcs[slot], semis[slot]

        def body(i, carry):
            src_col = pl.multiple_of((base + 128 * i) & (_T - 1), 128)
            dst_col = pl.multiple_of(128 * i, 128)
            pltpu.async_copy(
                x_hbm.at[0, pl.ds(o8, 8), pl.ds(src_col, 128)],
                xxc.at[:, pl.ds(dst_col, 128)],
                sem)
            return carry

        lax.fori_loop(0, _NSUB, body, 0)

    def wait_in(slot):
        xxc, sem = xxcs[slot], semis[slot]
        for _ in range(_NSUB):
            pltpu.make_async_copy(
                x_hbm.at[0, pl.ds(o8, 8), pl.ds(0, 128)],
                xxc.at[:, pl.ds(0, 128)],
                sem).wait()

    def drain_out(slot):
        for _ in range(2):
            pltpu.make_async_copy(
                rots[slot],
                out_hbm.at[0, pl.ds(o8, 8), pl.ds(0, _W)],
                semos[slot]).wait()

    def produce(batch, j2, slot):
        off = _sel_offset(batch)
        r = off & 127
        rl = r + lanes
        xxc, rot = xxcs[slot], rots[slot]

        def body(j):
            for u in range(8):
                col = j + u * _L
                for c in range(8):
                    rowv = jnp.full((_L,), c, jnp.int32)
                    vals = plsc.load_gather(xxc, [rowv, col + rl])
                    plsc.store_scatter(rot, [rowv, col + lanes], vals)

        plsc.parallel_loop(0, _W, 128)(body)
        t0 = pl.multiple_of((4 * q + 2 * j2) * 1024, _W)
        for half in range(2):
            dst_col = pl.multiple_of(t0 + half * _T, _W)
            pltpu.async_copy(
                rot, out_hbm.at[batch, pl.ds(o8, 8), pl.ds(dst_col, _W)],
                semos[slot])

    issue_in(jnp.int32(0), 0, 0)

    def step(m, carry):
        wait_in(0)
        issue_in(m, 1, 1)

        @pl.when(m > 0)
        def _():
            drain_out(0)

        produce(m, 0, 0)
        wait_in(1)

        @pl.when(m < _B - 1)
        def _():
            issue_in(m + 1, 0, 0)

        @pl.when(m > 0)
        def _():
            drain_out(1)

        produce(m, 1, 1)
        return carry

    lax.fori_loop(0, _B, step, 0)
    drain_out(0)
    drain_out(1)


_CALL = pl.kernel(
    _jitter_sc,
    out_type=jax.ShapeDtypeStruct((_B, _C, _OUT_T), jnp.float32),
    mesh=_MESH,
    scratch_types=[
        pltpu.VMEM((8, _NSUB * 128), jnp.float32),
        pltpu.VMEM((8, _NSUB * 128), jnp.float32),
        pltpu.VMEM((8, _W), jnp.float32),
        pltpu.VMEM((8, _W), jnp.float32),
        pltpu.SemaphoreType.DMA,
        pltpu.SemaphoreType.DMA,
        pltpu.SemaphoreType.DMA,
        pltpu.SemaphoreType.DMA,
    ],
    compiler_params=pltpu.CompilerParams(needs_layout_passes=False),
)


def kernel(x):
    return _CALL(x)
